# baseline (speedup 1.0000x reference)
import jax
import jax.numpy as jnp
from jax.experimental import pallas as pl
from jax.experimental.pallas import tpu as pltpu

BN_ROWS = 1600
NSEG = 10000

_SMEM = getattr(pltpu, "SMEM", None)
if _SMEM is None:
    _SMEM = pltpu.MemorySpace.SMEM


def _gram_kernel(x_ref, g_ref, s_ref):
    @pl.when(pl.program_id(0) == 0)
    def _():
        g_ref[...] = jnp.zeros_like(g_ref)
        s_ref[...] = jnp.zeros_like(s_ref)

    x = x_ref[...]
    g_ref[...] += jax.lax.dot_general(x, x, (((0,), (0,)), ((), ())),
                                      preferred_element_type=jnp.float32)
    s_ref[...] += jnp.sum(x, axis=0, keepdims=True)


def _layer_kernel(x_ref, a_ref, c_ref, y_ref, g_ref, s_ref):
    @pl.when(pl.program_id(0) == 0)
    def _():
        g_ref[...] = jnp.zeros_like(g_ref)
        s_ref[...] = jnp.zeros_like(s_ref)

    y = jnp.maximum(jnp.dot(x_ref[...], a_ref[...],
                            preferred_element_type=jnp.float32)
                    + c_ref[...], 0.0)
    y_ref[...] = y
    g_ref[...] += jax.lax.dot_general(y, y, (((0,), (0,)), ((), ())),
                                      preferred_element_type=jnp.float32)
    s_ref[...] += jnp.sum(y, axis=0, keepdims=True)


def _layer3_kernel(x_ref, a_ref, c_ref, ws_ref, bs_ref, y_ref, sc_ref):
    y = jnp.dot(x_ref[...], a_ref[...],
                preferred_element_type=jnp.float32) + c_ref[...]
    y_ref[...] = y
    sc_ref[...] = jnp.dot(y, ws_ref[...],
                          preferred_element_type=jnp.float32) + bs_ref[...]


def _segmax_kernel(inv_ref, sc_ref, out_ref):
    @pl.when(pl.program_id(0) == 0)
    def _():
        out_ref[...] = jnp.full_like(out_ref, -jnp.inf)

    def body(i, carry):
        iv = inv_ref[0, 0, i]
        row = sc_ref[pl.ds(i, 1), :]
        out_ref[pl.ds(iv, 1), :] = jnp.maximum(out_ref[pl.ds(iv, 1), :], row)
        return carry

    jax.lax.fori_loop(0, sc_ref.shape[0], body, 0)


def _scatter_kernel(inv_ref, sc_ref, x_ref, smax_ref, esum_ref, xe_ref):
    @pl.when(pl.program_id(0) == 0)
    def _():
        esum_ref[...] = jnp.zeros_like(esum_ref)
        xe_ref[...] = jnp.zeros_like(xe_ref)

    def body(i, carry):
        iv = inv_ref[0, 0, i]
        e = jnp.exp(sc_ref[pl.ds(i, 1), :] - smax_ref[pl.ds(iv, 1), :])
        esum_ref[pl.ds(iv, 1), :] += e
        xrow = x_ref[pl.ds(i, 1), :]
        contrib = e[0][:, None] * xrow
        xe_ref[pl.ds(iv, 1), :, :] += contrib[None]
        return carry

    jax.lax.fori_loop(0, sc_ref.shape[0], body, 0)


def _gather_kernel(idx_ref, xe_ref, esum_ref, h_ref, s_ref, q_ref):
    @pl.when(pl.program_id(0) == 0)
    def _():
        s_ref[...] = jnp.zeros_like(s_ref)
        q_ref[...] = jnp.zeros_like(q_ref)

    def body(i, carry):
        iv = idx_ref[0, 0, i]
        w = xe_ref[pl.ds(iv, 1), :, :][0]
        d = jnp.maximum(esum_ref[pl.ds(iv, 1), :], 0.5)
        hv = w * (1.0 / d[0][:, None])
        h_ref[pl.ds(i, 1), :, :] = hv[None]
        s_ref[...] += jnp.sum(hv, axis=0, keepdims=True)
        q_ref[...] += jnp.sum(hv * hv, axis=0, keepdims=True)
        return carry

    jax.lax.fori_loop(0, h_ref.shape[0], body, 0)


def _attn_kernel(inp_ref, x_ref, h_ref, sca_ref, shf_ref, wq_ref, bq_ref,
                 wk_ref, bk_ref, wv_ref, bv_ref, wo_ref, bo_ref, out_ref):
    d = x_ref.shape[1]
    nl = h_ref.shape[1]
    q = jnp.dot(x_ref[...], wq_ref[...],
                preferred_element_type=jnp.float32) + bq_ref[...]
    logits = []
    vs = []
    for l in range(nl):
        hs = h_ref[:, l, :] * sca_ref[...] + shf_ref[...]
        k = jnp.dot(hs, wk_ref[...],
                    preferred_element_type=jnp.float32) + bk_ref[...]
        v = jnp.dot(hs, wv_ref[...],
                    preferred_element_type=jnp.float32) + bv_ref[...]
        logits.append(jnp.sum(q * k, axis=1, keepdims=True))
        vs.append(v)
    lg = jnp.concatenate(logits, axis=1) * (1.0 / jnp.sqrt(jnp.float32(d)))
    mx = jnp.max(lg, axis=1, keepdims=True)
    ex = jnp.exp(lg - mx)
    a = ex / jnp.sum(ex, axis=1, keepdims=True)
    o = a[:, 0:1] * vs[0]
    for l in range(1, nl):
        o = o + a[:, l:l + 1] * vs[l]
    o = jnp.dot(o, wo_ref[...],
                preferred_element_type=jnp.float32) + bo_ref[...]
    out_ref[:, :d] = inp_ref[...]
    out_ref[:, d:] = o


def _bn_fold(gram, colsum, n, W, b, g, be):
    m = colsum[0] / n
    gn = gram / n
    mean = m @ W.T + b
    ey2 = jnp.sum((W @ gn) * W, axis=1) + 2.0 * b * (W @ m) + b * b
    var = ey2 - mean * mean
    scale = g / jnp.sqrt(var + 1e-3)
    shift = be - mean * scale
    A = W.T * scale[None, :]
    c = (b * scale + shift)[None, :]
    return A, c


def kernel(inp, inverse, topk_idx, W1, b1, g1, be1, W2, b2, g2, be2,
           W3, b3, g3, be3, Ws, bs, gn, bnb, Wqkv, bqkv, Wo, bo):
    n, d = inp.shape
    nl = Ws.shape[0]
    m = NSEG
    bnr = BN_ROWS
    nb = n // bnr
    f32 = jnp.float32

    grid1 = (nb,)
    row_spec = pl.BlockSpec((bnr, d), lambda i: (i, 0))
    full_mat = pl.BlockSpec((d, d), lambda i: (0, 0))
    full_row = pl.BlockSpec((1, d), lambda i: (0, 0))

    g0, s0 = pl.pallas_call(
        _gram_kernel,
        grid=grid1,
        in_specs=[row_spec],
        out_specs=[full_mat, full_row],
        out_shape=[jax.ShapeDtypeStruct((d, d), f32),
                   jax.ShapeDtypeStruct((1, d), f32)],
    )(inp)

    A1, c1 = _bn_fold(g0, s0, n, W1, b1, g1, be1)

    def run_layer(x, A, c):
        return pl.pallas_call(
            _layer_kernel,
            grid=grid1,
            in_specs=[row_spec, full_mat, full_row],
            out_specs=[row_spec, full_mat, full_row],
            out_shape=[jax.ShapeDtypeStruct((n, d), f32),
                       jax.ShapeDtypeStruct((d, d), f32),
                       jax.ShapeDtypeStruct((1, d), f32)],
        )(x, A, c)

    x1, g1m, s1 = run_layer(inp, A1, c1)
    A2, c2 = _bn_fold(g1m, s1, n, W2, b2, g2, be2)
    x2, g2m, s2 = run_layer(x1, A2, c2)
    A3, c3 = _bn_fold(g2m, s2, n, W3, b3, g3, be3)

    x3, scores = pl.pallas_call(
        _layer3_kernel,
        grid=grid1,
        in_specs=[row_spec, full_mat, full_row,
                  pl.BlockSpec((d, nl), lambda i: (0, 0)),
                  pl.BlockSpec((1, nl), lambda i: (0, 0))],
        out_specs=[row_spec, pl.BlockSpec((bnr, nl), lambda i: (i, 0))],
        out_shape=[jax.ShapeDtypeStruct((n, d), f32),
                   jax.ShapeDtypeStruct((n, nl), f32)],
    )(x2, A3, c3, Ws.T, bs[None, :])

    inv2 = inverse.astype(jnp.int32).reshape(nb, 1, bnr)
    idx2 = topk_idx[0].astype(jnp.int32).reshape(nb, 1, bnr)
    idx_spec = pl.BlockSpec((1, 1, bnr), lambda i: (i, 0, 0),
                            memory_space=_SMEM)
    sc_spec = pl.BlockSpec((bnr, nl), lambda i: (i, 0))
    seg_spec = pl.BlockSpec((m, nl), lambda i: (0, 0))
    seg3_spec = pl.BlockSpec((m, nl, d), lambda i: (0, 0, 0))

    smax = pl.pallas_call(
        _segmax_kernel,
        grid=grid1,
        in_specs=[idx_spec, sc_spec],
        out_specs=seg_spec,
        out_shape=jax.ShapeDtypeStruct((m, nl), f32),
    )(inv2, scores)

    esum, xe = pl.pallas_call(
        _scatter_kernel,
        grid=grid1,
        in_specs=[idx_spec, sc_spec, row_spec, seg_spec],
        out_specs=[seg_spec, seg3_spec],
        out_shape=[jax.ShapeDtypeStruct((m, nl), f32),
                   jax.ShapeDtypeStruct((m, nl, d), f32)],
    )(inv2, scores, x3, smax)

    h, hs_sum, hs_sq = pl.pallas_call(
        _gather_kernel,
        grid=grid1,
        in_specs=[idx_spec, seg3_spec, seg_spec],
        out_specs=[pl.BlockSpec((bnr, nl, d), lambda i: (i, 0, 0)),
                   full_row, full_row],
        out_shape=[jax.ShapeDtypeStruct((n, nl, d), f32),
                   jax.ShapeDtypeStruct((1, d), f32),
                   jax.ShapeDtypeStruct((1, d), f32)],
    )(idx2, xe, esum)

    cnt = jnp.float32(n * nl)
    meanN = hs_sum / cnt
    varN = hs_sq / cnt - meanN * meanN
    scaleN = gn[None, :] / jnp.sqrt(varN + 1e-3)
    shiftN = bnb[None, :] - meanN * scaleN

    wq = Wqkv[:d].T
    wk = Wqkv[d:2 * d].T
    wv = Wqkv[2 * d:].T
    bq = bqkv[None, :d]
    bk = bqkv[None, d:2 * d]
    bv = bqkv[None, 2 * d:]

    out = pl.pallas_call(
        _attn_kernel,
        grid=grid1,
        in_specs=[row_spec, row_spec,
                  pl.BlockSpec((bnr, nl, d), lambda i: (i, 0, 0)),
                  full_row, full_row,
                  full_mat, full_row, full_mat, full_row,
                  full_mat, full_row, full_mat, full_row],
        out_specs=pl.BlockSpec((bnr, 2 * d), lambda i: (i, 0)),
        out_shape=jax.ShapeDtypeStruct((n, 2 * d), f32),
    )(inp, x3, h, scaleN, shiftN, wq, bq, wk, bk, wv, bv, Wo.T, bo[None, :])

    return out


# speedup vs baseline: 3.8728x; 3.8728x over previous
import jax
import jax.numpy as jnp
from jax.experimental import pallas as pl
from jax.experimental.pallas import tpu as pltpu

BN_ROWS = 1600
NSEG = 10000

_SMEM = getattr(pltpu, "SMEM", None)
if _SMEM is None:
    _SMEM = pltpu.MemorySpace.SMEM


def _gram_kernel(x_ref, g_ref, s_ref):
    @pl.when(pl.program_id(0) == 0)
    def _():
        g_ref[...] = jnp.zeros_like(g_ref)
        s_ref[...] = jnp.zeros_like(s_ref)

    x = x_ref[...]
    g_ref[...] += jax.lax.dot_general(x, x, (((0,), (0,)), ((), ())),
                                      preferred_element_type=jnp.float32)
    s_ref[...] += jnp.sum(x, axis=0, keepdims=True)


def _layer_kernel(x_ref, a_ref, c_ref, y_ref, g_ref, s_ref):
    @pl.when(pl.program_id(0) == 0)
    def _():
        g_ref[...] = jnp.zeros_like(g_ref)
        s_ref[...] = jnp.zeros_like(s_ref)

    y = jnp.maximum(jnp.dot(x_ref[...], a_ref[...],
                            preferred_element_type=jnp.float32)
                    + c_ref[...], 0.0)
    y_ref[...] = y
    g_ref[...] += jax.lax.dot_general(y, y, (((0,), (0,)), ((), ())),
                                      preferred_element_type=jnp.float32)
    s_ref[...] += jnp.sum(y, axis=0, keepdims=True)


def _layer3_kernel(x_ref, a_ref, c_ref, ws_ref, bs_ref, y_ref, sc_ref):
    y = jnp.dot(x_ref[...], a_ref[...],
                preferred_element_type=jnp.float32) + c_ref[...]
    y_ref[...] = y
    sc_ref[...] = jnp.dot(y, ws_ref[...],
                          preferred_element_type=jnp.float32) + bs_ref[...]


def _segmax_kernel(inv_ref, sc_ref, out_ref):
    @pl.when(pl.program_id(0) == 0)
    def _():
        out_ref[...] = jnp.full_like(out_ref, -jnp.inf)

    def body(i, carry):
        iv = inv_ref[0, 0, i]
        row = sc_ref[pl.ds(i, 1), :]
        out_ref[pl.ds(iv, 1), :] = jnp.maximum(out_ref[pl.ds(iv, 1), :], row)
        return carry

    jax.lax.fori_loop(0, sc_ref.shape[0], body, 0, unroll=8)


def _scatter_kernel(inv_ref, sc_ref, x_ref, smax_ref, esum_ref, xe_ref):
    @pl.when(pl.program_id(0) == 0)
    def _():
        esum_ref[...] = jnp.zeros_like(esum_ref)
        xe_ref[...] = jnp.zeros_like(xe_ref)

    def body(i, carry):
        iv = inv_ref[0, 0, i]
        e = jnp.exp(sc_ref[pl.ds(i, 1), :] - smax_ref[pl.ds(iv, 1), :])
        esum_ref[pl.ds(iv, 1), :] += e
        xrow = x_ref[pl.ds(i, 1), :]
        contrib = e[0][:, None] * xrow
        xe_ref[pl.ds(iv, 1), :, :] += contrib[None]
        return carry

    jax.lax.fori_loop(0, sc_ref.shape[0], body, 0, unroll=8)


def _gather_kernel(idx_ref, xe_ref, esum_ref, h_ref, s_ref, q_ref):
    @pl.when(pl.program_id(0) == 0)
    def _():
        s_ref[...] = jnp.zeros_like(s_ref)
        q_ref[...] = jnp.zeros_like(q_ref)

    def body(i, carry):
        iv = idx_ref[0, 0, i]
        w = xe_ref[pl.ds(iv, 1), :, :][0]
        d = jnp.maximum(esum_ref[pl.ds(iv, 1), :], 0.5)
        hv = w * (1.0 / d[0][:, None])
        h_ref[pl.ds(i, 1), :, :] = hv[None]
        s_ref[...] += jnp.sum(hv, axis=0, keepdims=True)
        q_ref[...] += jnp.sum(hv * hv, axis=0, keepdims=True)
        return carry

    jax.lax.fori_loop(0, h_ref.shape[0], body, 0, unroll=8)


def _attn_kernel(inp_ref, x_ref, h_ref, sca_ref, shf_ref, wq_ref, bq_ref,
                 wk_ref, bk_ref, wv_ref, bv_ref, wo_ref, bo_ref, out_ref):
    d = x_ref.shape[1]
    nl = h_ref.shape[1]
    q = jnp.dot(x_ref[...], wq_ref[...],
                preferred_element_type=jnp.float32) + bq_ref[...]
    logits = []
    vs = []
    for l in range(nl):
        hs = h_ref[:, l, :] * sca_ref[...] + shf_ref[...]
        k = jnp.dot(hs, wk_ref[...],
                    preferred_element_type=jnp.float32) + bk_ref[...]
        v = jnp.dot(hs, wv_ref[...],
                    preferred_element_type=jnp.float32) + bv_ref[...]
        logits.append(jnp.sum(q * k, axis=1, keepdims=True))
        vs.append(v)
    lg = jnp.concatenate(logits, axis=1) * (1.0 / jnp.sqrt(jnp.float32(d)))
    mx = jnp.max(lg, axis=1, keepdims=True)
    ex = jnp.exp(lg - mx)
    a = ex / jnp.sum(ex, axis=1, keepdims=True)
    o = a[:, 0:1] * vs[0]
    for l in range(1, nl):
        o = o + a[:, l:l + 1] * vs[l]
    o = jnp.dot(o, wo_ref[...],
                preferred_element_type=jnp.float32) + bo_ref[...]
    out_ref[:, :d] = inp_ref[...]
    out_ref[:, d:] = o


def _bn_fold(gram, colsum, n, W, b, g, be):
    m = colsum[0] / n
    gn = gram / n
    mean = m @ W.T + b
    ey2 = jnp.sum((W @ gn) * W, axis=1) + 2.0 * b * (W @ m) + b * b
    var = ey2 - mean * mean
    scale = g / jnp.sqrt(var + 1e-3)
    shift = be - mean * scale
    A = W.T * scale[None, :]
    c = (b * scale + shift)[None, :]
    return A, c


def kernel(inp, inverse, topk_idx, W1, b1, g1, be1, W2, b2, g2, be2,
           W3, b3, g3, be3, Ws, bs, gn, bnb, Wqkv, bqkv, Wo, bo):
    n, d = inp.shape
    nl = Ws.shape[0]
    m = NSEG
    bnr = BN_ROWS
    nb = n // bnr
    f32 = jnp.float32

    grid1 = (nb,)
    row_spec = pl.BlockSpec((bnr, d), lambda i: (i, 0))
    full_mat = pl.BlockSpec((d, d), lambda i: (0, 0))
    full_row = pl.BlockSpec((1, d), lambda i: (0, 0))

    g0, s0 = pl.pallas_call(
        _gram_kernel,
        grid=grid1,
        in_specs=[row_spec],
        out_specs=[full_mat, full_row],
        out_shape=[jax.ShapeDtypeStruct((d, d), f32),
                   jax.ShapeDtypeStruct((1, d), f32)],
    )(inp)

    A1, c1 = _bn_fold(g0, s0, n, W1, b1, g1, be1)

    def run_layer(x, A, c):
        return pl.pallas_call(
            _layer_kernel,
            grid=grid1,
            in_specs=[row_spec, full_mat, full_row],
            out_specs=[row_spec, full_mat, full_row],
            out_shape=[jax.ShapeDtypeStruct((n, d), f32),
                       jax.ShapeDtypeStruct((d, d), f32),
                       jax.ShapeDtypeStruct((1, d), f32)],
        )(x, A, c)

    x1, g1m, s1 = run_layer(inp, A1, c1)
    A2, c2 = _bn_fold(g1m, s1, n, W2, b2, g2, be2)
    x2, g2m, s2 = run_layer(x1, A2, c2)
    A3, c3 = _bn_fold(g2m, s2, n, W3, b3, g3, be3)

    x3, scores = pl.pallas_call(
        _layer3_kernel,
        grid=grid1,
        in_specs=[row_spec, full_mat, full_row,
                  pl.BlockSpec((d, nl), lambda i: (0, 0)),
                  pl.BlockSpec((1, nl), lambda i: (0, 0))],
        out_specs=[row_spec, pl.BlockSpec((bnr, nl), lambda i: (i, 0))],
        out_shape=[jax.ShapeDtypeStruct((n, d), f32),
                   jax.ShapeDtypeStruct((n, nl), f32)],
    )(x2, A3, c3, Ws.T, bs[None, :])

    inv2 = inverse.astype(jnp.int32).reshape(nb, 1, bnr)
    idx2 = topk_idx[0].astype(jnp.int32).reshape(nb, 1, bnr)
    idx_spec = pl.BlockSpec((1, 1, bnr), lambda i: (i, 0, 0),
                            memory_space=_SMEM)
    sc_spec = pl.BlockSpec((bnr, nl), lambda i: (i, 0))
    seg_spec = pl.BlockSpec((m, nl), lambda i: (0, 0))
    seg3_spec = pl.BlockSpec((m, nl, d), lambda i: (0, 0, 0))

    smax = pl.pallas_call(
        _segmax_kernel,
        grid=grid1,
        in_specs=[idx_spec, sc_spec],
        out_specs=seg_spec,
        out_shape=jax.ShapeDtypeStruct((m, nl), f32),
    )(inv2, scores)

    esum, xe = pl.pallas_call(
        _scatter_kernel,
        grid=grid1,
        in_specs=[idx_spec, sc_spec, row_spec, seg_spec],
        out_specs=[seg_spec, seg3_spec],
        out_shape=[jax.ShapeDtypeStruct((m, nl), f32),
                   jax.ShapeDtypeStruct((m, nl, d), f32)],
    )(inv2, scores, x3, smax)

    h, hs_sum, hs_sq = pl.pallas_call(
        _gather_kernel,
        grid=grid1,
        in_specs=[idx_spec, seg3_spec, seg_spec],
        out_specs=[pl.BlockSpec((bnr, nl, d), lambda i: (i, 0, 0)),
                   full_row, full_row],
        out_shape=[jax.ShapeDtypeStruct((n, nl, d), f32),
                   jax.ShapeDtypeStruct((1, d), f32),
                   jax.ShapeDtypeStruct((1, d), f32)],
    )(idx2, xe, esum)

    cnt = jnp.float32(n * nl)
    meanN = hs_sum / cnt
    varN = hs_sq / cnt - meanN * meanN
    scaleN = gn[None, :] / jnp.sqrt(varN + 1e-3)
    shiftN = bnb[None, :] - meanN * scaleN

    wq = Wqkv[:d].T
    wk = Wqkv[d:2 * d].T
    wv = Wqkv[2 * d:].T
    bq = bqkv[None, :d]
    bk = bqkv[None, d:2 * d]
    bv = bqkv[None, 2 * d:]

    out = pl.pallas_call(
        _attn_kernel,
        grid=grid1,
        in_specs=[row_spec, row_spec,
                  pl.BlockSpec((bnr, nl, d), lambda i: (i, 0, 0)),
                  full_row, full_row,
                  full_mat, full_row, full_mat, full_row,
                  full_mat, full_row, full_mat, full_row],
        out_specs=pl.BlockSpec((bnr, 2 * d), lambda i: (i, 0)),
        out_shape=jax.ShapeDtypeStruct((n, 2 * d), f32),
    )(inp, x3, h, scaleN, shiftN, wq, bq, wk, bk, wv, bv, Wo.T, bo[None, :])

    return out
